# baseline (speedup 1.0000x reference)
import jax
import jax.numpy as jnp
from jax.experimental import pallas as pl


def _body(feat_ref, coordsT_ref, hb1_ref, zr_ref, hW1_ref, hW2p_ref, hb2p_ref,
          aW1_ref, ab1_ref, aW2p_ref, ab2p_ref,
          xt_ref, a_ref):
    f = feat_ref[...]
    z = jnp.dot(f.astype(jnp.bfloat16), hW1_ref[...].astype(jnp.bfloat16),
                preferred_element_type=jnp.float32)
    z = z + hb1_ref[...] + zr_ref[0, 0]
    h = jnp.where(z > 0, z, 0.02 * z)
    hb = h.astype(jnp.bfloat16)
    he = jnp.dot(hb, hW2p_ref[...].astype(jnp.bfloat16),
                 preferred_element_type=jnp.float32)

    heatT = jnp.transpose(he[:, 0:1]) + hb2p_ref[0, 0]
    m = jnp.max(heatT)
    eT = jnp.exp(heatT - m)
    ssum = jnp.sum(eT)
    v = jnp.dot(eT.astype(jnp.bfloat16), hb,
                preferred_element_type=jnp.float32)
    ve = jnp.dot(v.astype(jnp.bfloat16), hW2p_ref[...].astype(jnp.bfloat16),
                 preferred_element_type=jnp.float32)
    wc = jnp.sum(coordsT_ref[...] * eT, axis=1, keepdims=True)
    xt = (jnp.transpose(wc) + ve[:, 1:4]) / ssum + hb2p_ref[:, 1:4]
    xt_ref[0, :, :] = xt

    pc = jnp.max(f, axis=0, keepdims=True)
    act = jnp.dot(pc.astype(jnp.bfloat16), aW1_ref[...].astype(jnp.bfloat16),
                  preferred_element_type=jnp.float32)
    act = act + ab1_ref[...]
    act = jnp.where(act > 0, act, 0.02 * act)
    a = jnp.dot(act.astype(jnp.bfloat16), aW2p_ref[...].astype(jnp.bfloat16),
                preferred_element_type=jnp.float32)
    a_ref[0, :, :] = a + ab2p_ref[...]


def kernel(feat, npoints_in_batch, coords, hW1, hb1, hW2, hb2, aW1, ab1, aW2, ab2):
    N, D = feat.shape
    S = 2048
    B = N // S
    OUT = aW2.shape[1]
    EB = (OUT - 1) // 3
    OUTP = 256
    zr = ((jnp.asarray(npoints_in_batch) - S).astype(feat.dtype)).reshape(1, 1)

    coordsT = coords.T
    hW2p = jnp.pad(hW2, ((0, 0), (0, 128 - hW2.shape[1])))
    hb2p = jnp.pad(hb2, (0, 128 - hb2.shape[0])).reshape(1, 128)
    aW2p = jnp.pad(aW2, ((0, 0), (0, OUTP - OUT)))
    ab2p = jnp.pad(ab2, (0, OUTP - OUT)).reshape(1, OUTP)

    xt3, a3 = pl.pallas_call(
        _body,
        grid=(B,),
        in_specs=[
            pl.BlockSpec((S, D), lambda b: (b, 0)),
            pl.BlockSpec((3, S), lambda b: (0, b)),
            pl.BlockSpec((1, D), lambda b: (0, 0)),
            pl.BlockSpec((1, 1), lambda b: (0, 0)),
            pl.BlockSpec((D, D), lambda b: (0, 0)),
            pl.BlockSpec((D, 128), lambda b: (0, 0)),
            pl.BlockSpec((1, 128), lambda b: (0, 0)),
            pl.BlockSpec((D, D), lambda b: (0, 0)),
            pl.BlockSpec((1, D), lambda b: (0, 0)),
            pl.BlockSpec((D, OUTP), lambda b: (0, 0)),
            pl.BlockSpec((1, OUTP), lambda b: (0, 0)),
        ],
        out_specs=[
            pl.BlockSpec((1, 1, 3), lambda b: (b, 0, 0)),
            pl.BlockSpec((1, 1, OUTP), lambda b: (b, 0, 0)),
        ],
        out_shape=[
            jax.ShapeDtypeStruct((B, 1, 3), feat.dtype),
            jax.ShapeDtypeStruct((B, 1, OUTP), feat.dtype),
        ],
    )(feat, coordsT, hb1.reshape(1, D), zr, hW1, hW2p, hb2p, aW1,
      ab1.reshape(1, D), aW2p, ab2p)

    xt = xt3.reshape(B, 3)
    a = a3.reshape(B, OUTP)
    xr = a[:, :EB * 3].reshape(-1, EB, 3)
    xo = a[:, OUT - 1]
    return (xt, xr, xo)


# speedup vs baseline: 1.1597x; 1.1597x over previous
import jax
import jax.numpy as jnp
from jax.experimental import pallas as pl


def _body(feat_ref, coordsT_ref, hb1_ref, zr_ref, hW1_ref, hW2p_ref, hb2p_ref,
          aW1_ref, ab1_ref, aW2p_ref, ab2p_ref,
          xt_ref, a_ref):
    f = feat_ref[...]
    z = jnp.dot(f.astype(jnp.bfloat16), hW1_ref[...].astype(jnp.bfloat16),
                preferred_element_type=jnp.float32)
    z = z + hb1_ref[...] + zr_ref[0, 0]
    h = jnp.where(z > 0, z, 0.02 * z)
    hb = h.astype(jnp.bfloat16)
    he = jnp.dot(hb, hW2p_ref[...].astype(jnp.bfloat16),
                 preferred_element_type=jnp.float32)

    heat = he[:, 0:1] + hb2p_ref[0, 0]
    m = jnp.max(heat)
    e = jnp.exp(heat - m)
    ssum = jnp.sum(e)
    eT = jnp.transpose(e)
    v = jnp.dot(eT.astype(jnp.bfloat16), hb,
                preferred_element_type=jnp.float32)
    ve = jnp.dot(v.astype(jnp.bfloat16), hW2p_ref[...].astype(jnp.bfloat16),
                 preferred_element_type=jnp.float32)
    wc = jnp.sum(coordsT_ref[...] * eT, axis=1, keepdims=True)
    xt = (jnp.transpose(wc) + ve[:, 1:4]) / ssum + hb2p_ref[:, 1:4]
    xt_ref[0, :, :] = xt

    pc = jnp.max(f, axis=0, keepdims=True)
    act = jnp.dot(pc.astype(jnp.bfloat16), aW1_ref[...].astype(jnp.bfloat16),
                  preferred_element_type=jnp.float32)
    act = act + ab1_ref[...]
    act = jnp.where(act > 0, act, 0.02 * act)
    a = jnp.dot(act.astype(jnp.bfloat16), aW2p_ref[...].astype(jnp.bfloat16),
                preferred_element_type=jnp.float32)
    a_ref[0, :, :] = a + ab2p_ref[...]


def kernel(feat, npoints_in_batch, coords, hW1, hb1, hW2, hb2, aW1, ab1, aW2, ab2):
    N, D = feat.shape
    S = 2048
    B = N // S
    OUT = aW2.shape[1]
    EB = (OUT - 1) // 3
    OUTP = 256
    zr = ((jnp.asarray(npoints_in_batch) - S).astype(feat.dtype)).reshape(1, 1)

    coordsT = coords.T
    hW2p = jnp.pad(hW2, ((0, 0), (0, 128 - hW2.shape[1])))
    hb2p = jnp.pad(hb2, (0, 128 - hb2.shape[0])).reshape(1, 128)
    aW2p = jnp.pad(aW2, ((0, 0), (0, OUTP - OUT)))
    ab2p = jnp.pad(ab2, (0, OUTP - OUT)).reshape(1, OUTP)

    xt3, a3 = pl.pallas_call(
        _body,
        grid=(B,),
        in_specs=[
            pl.BlockSpec((S, D), lambda b: (b, 0)),
            pl.BlockSpec((3, S), lambda b: (0, b)),
            pl.BlockSpec((1, D), lambda b: (0, 0)),
            pl.BlockSpec((1, 1), lambda b: (0, 0)),
            pl.BlockSpec((D, D), lambda b: (0, 0)),
            pl.BlockSpec((D, 128), lambda b: (0, 0)),
            pl.BlockSpec((1, 128), lambda b: (0, 0)),
            pl.BlockSpec((D, D), lambda b: (0, 0)),
            pl.BlockSpec((1, D), lambda b: (0, 0)),
            pl.BlockSpec((D, OUTP), lambda b: (0, 0)),
            pl.BlockSpec((1, OUTP), lambda b: (0, 0)),
        ],
        out_specs=[
            pl.BlockSpec((1, 1, 3), lambda b: (b, 0, 0)),
            pl.BlockSpec((1, 1, OUTP), lambda b: (b, 0, 0)),
        ],
        out_shape=[
            jax.ShapeDtypeStruct((B, 1, 3), feat.dtype),
            jax.ShapeDtypeStruct((B, 1, OUTP), feat.dtype),
        ],
    )(feat, coordsT, hb1.reshape(1, D), zr, hW1, hW2p, hb2p, aW1,
      ab1.reshape(1, D), aW2p, ab2p)

    xt = xt3.reshape(B, 3)
    a = a3.reshape(B, OUTP)
    xr = a[:, :EB * 3].reshape(-1, EB, 3)
    xo = a[:, OUT - 1]
    return (xt, xr, xo)
